# baseline (speedup 1.0000x reference)
import functools

import jax
import jax.numpy as jnp
from jax import lax
from jax.experimental import pallas as pl
from jax.experimental.pallas import tpu as pltpu
from jax.experimental.pallas import tpu_sc as plsc

NCAM = 6
QLEN = 4096
DM = 256
NH = 8
DH = 32
NPT = 8
ZC = 4
FH, FW = 32, 88
HW = FH * FW
TROWS = NCAM * HW * NH
NROWS = NCAM * QLEN
NCTR = 32

QB = 512
NW = 32
RPW = NROWS // NW
NQC = 12
NCHUNK = RPW // NQC
CE = NQC * 256


def _value_body(feat_ref, lvl_ref, cam_ref, w_ref, b_ref, out_ref):
    x = feat_ref[...] + lvl_ref[...] + cam_ref[...]
    v = lax.dot_general(x, w_ref[...], (((0,), (0,)), ((), ())),
                        preferred_element_type=jnp.float32) + b_ref[...]
    out_ref[...] = v.astype(jnp.bfloat16)


def _value_call(feat, lvl_emb, cam_emb, value_W, value_b):
    cam3 = cam_emb.reshape(NCAM, DM, 1)
    lvl3 = lvl_emb.reshape(1, DM, 1)
    return pl.pallas_call(
        _value_body,
        grid=(NCAM,),
        in_specs=[
            pl.BlockSpec((None, DM, HW), lambda c: (c, 0, 0)),
            pl.BlockSpec((None, DM, 1), lambda c: (0, 0, 0)),
            pl.BlockSpec((None, DM, 1), lambda c: (c, 0, 0)),
            pl.BlockSpec((DM, DM), lambda c: (0, 0)),
            pl.BlockSpec((1, DM), lambda c: (0, 0)),
        ],
        out_specs=pl.BlockSpec((None, HW, DM), lambda c: (c, 0, 0)),
        out_shape=jax.ShapeDtypeStruct((NCAM, HW, DM), jnp.bfloat16),
    )(feat, lvl3, cam3, value_W, value_b)


def _prep_body(q_ref, pos_ref, ow_ref, ob_ref, aw_ref, ab_ref, rx_ref, ry_ref,
               idx_ref, wgt_ref):
    c = pl.program_id(0)
    qp = q_ref[...] + pos_ref[...]
    off = jnp.dot(qp, ow_ref[...], preferred_element_type=jnp.float32) + ob_ref[...]
    logits = jnp.dot(qp, aw_ref[...], preferred_element_type=jnp.float32) + ab_ref[...]
    m = jnp.max(logits, axis=1, keepdims=True)
    t = jnp.exp(logits - m)
    i64 = lax.broadcasted_iota(jnp.int32, (64, 64), 0)
    j64 = lax.broadcasted_iota(jnp.int32, (64, 64), 1)
    g = ((i64 // NPT) == (j64 // NPT)).astype(jnp.float32)
    s = jnp.dot(t, g, preferred_element_type=jnp.float32)
    attw = t / s

    x = rx_ref[...] * FW + off[:, :64] - 0.5
    y = ry_ref[...] * FH + off[:, 64:] - 0.5
    x0f = jnp.floor(x)
    y0f = jnp.floor(y)
    lw = x - x0f
    lh = y - y0f
    x0 = x0f.astype(jnp.int32)
    y0 = y0f.astype(jnp.int32)
    x1 = x0 + 1
    y1 = y0 + 1
    vx0 = ((x0 >= 0) & (x0 < FW)).astype(jnp.float32)
    vx1 = ((x1 >= 0) & (x1 < FW)).astype(jnp.float32)
    vy0 = ((y0 >= 0) & (y0 < FH)).astype(jnp.float32)
    vy1 = ((y1 >= 0) & (y1 < FH)).astype(jnp.float32)
    cx0 = jnp.clip(x0, 0, FW - 1)
    cx1 = jnp.clip(x1, 0, FW - 1)
    cy0 = jnp.clip(y0, 0, FH - 1)
    cy1 = jnp.clip(y1, 0, FH - 1)
    h_lane = lax.broadcasted_iota(jnp.int32, (QB, 64), 1) // NPT
    base = c * (HW * NH) + h_lane

    def rowid(cy, cx):
        return base + (cy * FW + cx) * NH

    w00 = attw * (1 - lw) * (1 - lh) * (vx0 * vy0)
    w01 = attw * lw * (1 - lh) * (vx1 * vy0)
    w10 = attw * (1 - lw) * lh * (vx0 * vy1)
    w11 = attw * lw * lh * (vx1 * vy1)
    idx_ref[...] = jnp.concatenate(
        [rowid(cy0, cx0), rowid(cy0, cx1), rowid(cy1, cx0), rowid(cy1, cx1)], axis=1)
    wgt_ref[...] = jnp.concatenate([w00, w01, w10, w11], axis=1)


def _prep_call(q6, pos, ow_p, ob_p, attw_W, attw_b, rx64, ry64):
    nqb = QLEN // QB
    return pl.pallas_call(
        _prep_body,
        grid=(NCAM, nqb),
        in_specs=[
            pl.BlockSpec((None, QB, DM), lambda c, q: (c, q, 0)),
            pl.BlockSpec((QB, DM), lambda c, q: (q, 0)),
            pl.BlockSpec((DM, 2 * NH * NPT), lambda c, q: (0, 0)),
            pl.BlockSpec((1, 2 * NH * NPT), lambda c, q: (0, 0)),
            pl.BlockSpec((DM, NH * NPT), lambda c, q: (0, 0)),
            pl.BlockSpec((1, NH * NPT), lambda c, q: (0, 0)),
            pl.BlockSpec((None, QB, 64), lambda c, q: (c, q, 0)),
            pl.BlockSpec((None, QB, 64), lambda c, q: (c, q, 0)),
        ],
        out_specs=[
            pl.BlockSpec((None, QB, 256), lambda c, q: (c, q, 0)),
            pl.BlockSpec((None, QB, 256), lambda c, q: (c, q, 0)),
        ],
        out_shape=[
            jax.ShapeDtypeStruct((NCAM, QLEN, 256), jnp.int32),
            jax.ShapeDtypeStruct((NCAM, QLEN, 256), jnp.float32),
        ],
    )(q6, pos, ow_p, ob_p, attw_W, attw_b, rx64, ry64)


def _sc_body(table_hbm, idx_hbm, wgt_hbm, out_hbm,
             idx_v0, idx_v1, wgt_v0, wgt_v1, rows_v0, rows_v1, out_v,
             sem0, sem1, isem0, isem1, wsem0, wsem1):
    cid = lax.axis_index("c")
    sid = lax.axis_index("s")
    wid = sid * 2 + cid
    nrow = CE // 128
    idx_b = (idx_v0, idx_v1)
    wgt_b = (wgt_v0, wgt_v1)
    rows_b = (rows_v0, rows_v1)
    sem_b = (sem0, sem1)
    isem_b = (isem0, isem1)
    wsem_b = (wsem0, wsem1)
    base_row = wid * RPW * 2

    def idx_cp(t, b):
        rb = base_row + t * (NQC * 2)
        return pltpu.make_async_copy(
            idx_hbm.at[pl.ds(rb, nrow)], idx_b[b], isem_b[b])

    def wgt_cp(t, b):
        rb = base_row + t * (NQC * 2)
        return pltpu.make_async_copy(
            wgt_hbm.at[pl.ds(rb, nrow)], wgt_b[b], wsem_b[b])

    def fire_gathers(b):
        for i in range(nrow):
            pltpu.make_async_copy(
                table_hbm.at[idx_b[b].at[i]],
                rows_b[b].at[pl.ds(i * 128, 128)], sem_b[b]).start()

    def wait_rows(b):
        for i in range(nrow):
            pltpu.make_async_copy(
                table_hbm.at[idx_b[b].at[i]],
                rows_b[b].at[pl.ds(i * 128, 128)], sem_b[b]).wait()

    def compute(b):
        rows_v = rows_b[b]
        wgt_v = wgt_b[b]
        ob = b * NQC * NH

        def obody(ql, carry):
            qe = ql * 256
            for hh in range(NH):
                parts = []
                for k in range(4):
                    woff = k * 64 + hh * 8
                    cs = (woff & 127) & ~15
                    ws = (woff & 127) - cs
                    wv = wgt_v[ql * 2 + (woff >> 7), pl.ds(cs, 16)]
                    a0 = a1 = None
                    for p in range(8):
                        w = wv[ws + p]
                        rv = rows_v[qe + woff + p, :]
                        v0, v1 = plsc.unpack(
                            rv, format=plsc.PackFormat.INTERLEAVED,
                            preferred_element_type=jnp.float32)
                        a0 = w * v0 if a0 is None else a0 + w * v0
                        a1 = w * v1 if a1 is None else a1 + w * v1
                    parts.append((a0, a1))
                acc0 = (parts[0][0] + parts[1][0]) + (parts[2][0] + parts[3][0])
                acc1 = (parts[0][1] + parts[1][1]) + (parts[2][1] + parts[3][1])
                out_v[ob + ql * NH + hh, pl.ds(0, 16)] = acc0
                out_v[ob + ql * NH + hh, pl.ds(16, 16)] = acc1
            return carry

        lax.fori_loop(0, NQC, obody, 0)

    idx_cp(0, 0).start()
    wgt_cp(0, 0).start()
    idx_cp(0, 0).wait()
    fire_gathers(0)
    idx_cp(1, 1).start()
    wgt_cp(1, 1).start()

    def pair(i, carry):
        t = i * 2
        idx_cp(t + 1, 1).wait()
        fire_gathers(1)
        wait_rows(0)

        @pl.when(t + 2 < NCHUNK)
        def _():
            idx_cp(t + 2, 0).start()

        wgt_cp(t, 0).wait()
        compute(0)

        @pl.when(t + 2 < NCHUNK)
        def _():
            wgt_cp(t + 2, 0).start()

        @pl.when(t + 2 < NCHUNK)
        def _():
            idx_cp(t + 2, 0).wait()
            fire_gathers(0)

        wait_rows(1)

        @pl.when(t + 3 < NCHUNK)
        def _():
            idx_cp(t + 3, 1).start()

        wgt_cp(t + 1, 1).wait()
        compute(1)
        orb = (wid * RPW + t * NQC) * NH
        pltpu.sync_copy(out_v, out_hbm.at[pl.ds(orb, 2 * NQC * NH)])

        @pl.when(t + 3 < NCHUNK)
        def _():
            wgt_cp(t + 3, 1).start()

        return carry

    lax.fori_loop(0, NCHUNK // 2, pair, 0)


def _sc_call(table, idx_flat, wgt_flat):
    mesh = plsc.VectorSubcoreMesh(core_axis_name="c", subcore_axis_name="s")
    idx2 = idx_flat.reshape(-1, 128)
    wgt2 = wgt_flat.reshape(-1, 128)
    fn = functools.partial(
        pl.kernel,
        mesh=mesh,
        out_type=jax.ShapeDtypeStruct((NROWS * NH, DH), jnp.float32),
        compiler_params=pltpu.CompilerParams(
            needs_layout_passes=False, use_tc_tiling_on_sc=False),
        scratch_types=[
            pltpu.VMEM((CE // 128, 128), jnp.int32),
            pltpu.VMEM((CE // 128, 128), jnp.int32),
            pltpu.VMEM((CE // 128, 128), jnp.float32),
            pltpu.VMEM((CE // 128, 128), jnp.float32),
            pltpu.VMEM((CE, DH), jnp.bfloat16),
            pltpu.VMEM((CE, DH), jnp.bfloat16),
            pltpu.VMEM((2 * NQC * NH, DH), jnp.float32),
            pltpu.SemaphoreType.DMA,
            pltpu.SemaphoreType.DMA,
            pltpu.SemaphoreType.DMA,
            pltpu.SemaphoreType.DMA,
            pltpu.SemaphoreType.DMA,
            pltpu.SemaphoreType.DMA,
        ],
    )(_sc_body)
    return fn(table, idx2, wgt2)


def _fuse_body(attn_ref, mask_ref, w_ref, b_ref, out_ref):
    flags = (jnp.sum(mask_ref[...], axis=2) > 0).astype(jnp.float32)
    slots = jnp.sum(attn_ref[...] * flags[:, :, None], axis=0)
    cnt = jnp.maximum(jnp.sum(flags, axis=0), 1.0)
    slots = slots / cnt[:, None]
    out_ref[...] = (
        jnp.dot(slots, w_ref[...], preferred_element_type=jnp.float32) + b_ref[...]
    )


def _fuse_call(attn, maskf, out_W, out_b):
    nqb = QLEN // QB
    return pl.pallas_call(
        _fuse_body,
        grid=(nqb,),
        in_specs=[
            pl.BlockSpec((NCAM, QB, DM), lambda q: (0, q, 0)),
            pl.BlockSpec((NCAM, QB, ZC), lambda q: (0, q, 0)),
            pl.BlockSpec((DM, DM), lambda q: (0, 0)),
            pl.BlockSpec((1, DM), lambda q: (0, 0)),
        ],
        out_specs=pl.BlockSpec((QB, DM), lambda q: (q, 0)),
        out_shape=jax.ShapeDtypeStruct((QLEN, DM), jnp.float32),
    )(attn, maskf, out_W, out_b)


def kernel(queries, pos_emb, lvl_emb, cam_emb, features, reference_points_3D,
           bev_mask, value_W, value_b, off_W, off_b, attw_W, attw_b,
           out_W, out_b):
    dperm = jnp.stack([jnp.arange(16), jnp.arange(16) + 16], axis=1).reshape(32)
    vw_p = value_W.reshape(DM, NH, DH)[:, :, dperm].reshape(DM, DM)
    vb_p = value_b.reshape(NH, DH)[:, dperm].reshape(1, DM)
    value = _value_call(features.reshape(NCAM, DM, HW), lvl_emb, cam_emb,
                        vw_p, vb_p)

    ow_p = off_W.reshape(DM, NH, NPT, 2).transpose(0, 3, 1, 2).reshape(DM, 128)
    ob_p = off_b.reshape(NH, NPT, 2).transpose(2, 0, 1).reshape(1, 128)
    q6 = queries.reshape(NCAM, QLEN, DM)
    pos = pos_emb.reshape(QLEN, DM)
    r = reference_points_3D.reshape(NCAM, QLEN, ZC, 2)
    rx64 = jnp.broadcast_to(
        r[:, :, None, None, :, 0], (NCAM, QLEN, NH, 2, ZC)).reshape(NCAM, QLEN, 64)
    ry64 = jnp.broadcast_to(
        r[:, :, None, None, :, 1], (NCAM, QLEN, NH, 2, ZC)).reshape(NCAM, QLEN, 64)
    idx, wgt = _prep_call(q6, pos, ow_p, ob_p, attw_W, attw_b.reshape(1, 64),
                          rx64, ry64)

    table = value.reshape(TROWS, DH)
    attn_flat = _sc_call(table, idx.reshape(-1), wgt.reshape(-1))
    attn = attn_flat.reshape(NCAM, QLEN, DM)

    maskf = bev_mask.reshape(NCAM, QLEN, ZC).astype(jnp.float32)
    out = _fuse_call(attn, maskf, out_W, out_b.reshape(1, DM))
    return out.reshape(1, QLEN, DM)


# speedup vs baseline: 1.0314x; 1.0314x over previous
import functools

import jax
import jax.numpy as jnp
from jax import lax
from jax.experimental import pallas as pl
from jax.experimental.pallas import tpu as pltpu
from jax.experimental.pallas import tpu_sc as plsc

NCAM = 6
QLEN = 4096
DM = 256
NH = 8
DH = 32
NPT = 8
ZC = 4
FH, FW = 32, 88
HW = FH * FW
TROWS = NCAM * HW * NH
NROWS = NCAM * QLEN
NCTR = 32

QB = 512
NW = 32
RPW = NROWS // NW
NQC = 12
NCHUNK = RPW // NQC
CE = NQC * 256


def _value_body(feat_ref, lvl_ref, cam_ref, w_ref, b_ref, out_ref):
    x = feat_ref[...] + lvl_ref[...] + cam_ref[...]
    v = jnp.dot(x, w_ref[...], preferred_element_type=jnp.float32) + b_ref[...]
    out_ref[...] = v.astype(jnp.bfloat16)


def _value_call(feat_t, lvl_emb, cam_emb, value_W, value_b):
    cam3 = cam_emb.reshape(NCAM, 1, DM)
    return pl.pallas_call(
        _value_body,
        grid=(NCAM,),
        in_specs=[
            pl.BlockSpec((None, HW, DM), lambda c: (c, 0, 0)),
            pl.BlockSpec((1, DM), lambda c: (0, 0)),
            pl.BlockSpec((None, 1, DM), lambda c: (c, 0, 0)),
            pl.BlockSpec((DM, DM), lambda c: (0, 0)),
            pl.BlockSpec((1, DM), lambda c: (0, 0)),
        ],
        out_specs=pl.BlockSpec((None, HW, DM), lambda c: (c, 0, 0)),
        out_shape=jax.ShapeDtypeStruct((NCAM, HW, DM), jnp.bfloat16),
    )(feat_t, lvl_emb, cam3, value_W, value_b)


def _prep_body(q_ref, pos_ref, ow_ref, ob_ref, aw_ref, ab_ref, rx_ref, ry_ref,
               idx_ref, wgt_ref):
    c = pl.program_id(0)
    qp = q_ref[...] + pos_ref[...]
    off = jnp.dot(qp, ow_ref[...], preferred_element_type=jnp.float32) + ob_ref[...]
    logits = jnp.dot(qp, aw_ref[...], preferred_element_type=jnp.float32) + ab_ref[...]
    m = jnp.max(logits, axis=1, keepdims=True)
    t = jnp.exp(logits - m)
    i64 = lax.broadcasted_iota(jnp.int32, (64, 64), 0)
    j64 = lax.broadcasted_iota(jnp.int32, (64, 64), 1)
    g = ((i64 // NPT) == (j64 // NPT)).astype(jnp.float32)
    s = jnp.dot(t, g, preferred_element_type=jnp.float32)
    attw = t / s

    x = rx_ref[...] * FW + off[:, :64] - 0.5
    y = ry_ref[...] * FH + off[:, 64:] - 0.5
    x0f = jnp.floor(x)
    y0f = jnp.floor(y)
    lw = x - x0f
    lh = y - y0f
    x0 = x0f.astype(jnp.int32)
    y0 = y0f.astype(jnp.int32)
    x1 = x0 + 1
    y1 = y0 + 1
    vx0 = ((x0 >= 0) & (x0 < FW)).astype(jnp.float32)
    vx1 = ((x1 >= 0) & (x1 < FW)).astype(jnp.float32)
    vy0 = ((y0 >= 0) & (y0 < FH)).astype(jnp.float32)
    vy1 = ((y1 >= 0) & (y1 < FH)).astype(jnp.float32)
    cx0 = jnp.clip(x0, 0, FW - 1)
    cx1 = jnp.clip(x1, 0, FW - 1)
    cy0 = jnp.clip(y0, 0, FH - 1)
    cy1 = jnp.clip(y1, 0, FH - 1)
    h_lane = lax.broadcasted_iota(jnp.int32, (QB, 64), 1) // NPT
    base = c * (HW * NH) + h_lane

    def rowid(cy, cx):
        return base + (cy * FW + cx) * NH

    w00 = attw * (1 - lw) * (1 - lh) * (vx0 * vy0)
    w01 = attw * lw * (1 - lh) * (vx1 * vy0)
    w10 = attw * (1 - lw) * lh * (vx0 * vy1)
    w11 = attw * lw * lh * (vx1 * vy1)
    idx_ref[...] = jnp.concatenate(
        [rowid(cy0, cx0), rowid(cy0, cx1), rowid(cy1, cx0), rowid(cy1, cx1)], axis=1)
    wgt_ref[...] = jnp.concatenate([w00, w01, w10, w11], axis=1)


def _prep_call(q6, pos, ow_p, ob_p, attw_W, attw_b, rx64, ry64):
    nqb = QLEN // QB
    return pl.pallas_call(
        _prep_body,
        grid=(NCAM, nqb),
        in_specs=[
            pl.BlockSpec((None, QB, DM), lambda c, q: (c, q, 0)),
            pl.BlockSpec((QB, DM), lambda c, q: (q, 0)),
            pl.BlockSpec((DM, 2 * NH * NPT), lambda c, q: (0, 0)),
            pl.BlockSpec((1, 2 * NH * NPT), lambda c, q: (0, 0)),
            pl.BlockSpec((DM, NH * NPT), lambda c, q: (0, 0)),
            pl.BlockSpec((1, NH * NPT), lambda c, q: (0, 0)),
            pl.BlockSpec((None, QB, 64), lambda c, q: (c, q, 0)),
            pl.BlockSpec((None, QB, 64), lambda c, q: (c, q, 0)),
        ],
        out_specs=[
            pl.BlockSpec((None, QB, 256), lambda c, q: (c, q, 0)),
            pl.BlockSpec((None, QB, 256), lambda c, q: (c, q, 0)),
        ],
        out_shape=[
            jax.ShapeDtypeStruct((NCAM, QLEN, 256), jnp.int32),
            jax.ShapeDtypeStruct((NCAM, QLEN, 256), jnp.float32),
        ],
    )(q6, pos, ow_p, ob_p, attw_W, attw_b, rx64, ry64)


def _sc_body(table_hbm, idx_hbm, wgt_hbm, out_hbm,
             idx_v0, idx_v1, wgt_v0, wgt_v1, rows_v0, rows_v1, out_v,
             sem0, sem1, isem0, isem1, wsem0, wsem1):
    cid = lax.axis_index("c")
    sid = lax.axis_index("s")
    wid = sid * 2 + cid
    nrow = CE // 128
    idx_b = (idx_v0, idx_v1)
    wgt_b = (wgt_v0, wgt_v1)
    rows_b = (rows_v0, rows_v1)
    sem_b = (sem0, sem1)
    isem_b = (isem0, isem1)
    wsem_b = (wsem0, wsem1)
    base_row = wid * RPW * 2

    def idx_cp(t, b):
        rb = base_row + t * (NQC * 2)
        return pltpu.make_async_copy(
            idx_hbm.at[pl.ds(rb, nrow)], idx_b[b], isem_b[b])

    def wgt_cp(t, b):
        rb = base_row + t * (NQC * 2)
        return pltpu.make_async_copy(
            wgt_hbm.at[pl.ds(rb, nrow)], wgt_b[b], wsem_b[b])

    def fire_gathers(b):
        for i in range(nrow):
            pltpu.make_async_copy(
                table_hbm.at[idx_b[b].at[i]],
                rows_b[b].at[pl.ds(i * 128, 128)], sem_b[b]).start()

    def wait_rows(b):
        for i in range(nrow):
            pltpu.make_async_copy(
                table_hbm.at[idx_b[b].at[i]],
                rows_b[b].at[pl.ds(i * 128, 128)], sem_b[b]).wait()

    def compute(b):
        rows_v = rows_b[b]
        wgt_v = wgt_b[b]
        ob = b * NQC * NH

        def obody(ql, carry):
            qe = ql * 256
            for hh in range(NH):
                parts = []
                for k in range(4):
                    woff = k * 64 + hh * 8
                    cs = (woff & 127) & ~15
                    ws = (woff & 127) - cs
                    wv = wgt_v[ql * 2 + (woff >> 7), pl.ds(cs, 16)]
                    a0 = a1 = None
                    for p in range(8):
                        w = wv[ws + p]
                        rv = rows_v[qe + woff + p, :]
                        v0, v1 = plsc.unpack(
                            rv, format=plsc.PackFormat.INTERLEAVED,
                            preferred_element_type=jnp.float32)
                        a0 = w * v0 if a0 is None else a0 + w * v0
                        a1 = w * v1 if a1 is None else a1 + w * v1
                    parts.append((a0, a1))
                acc0 = (parts[0][0] + parts[1][0]) + (parts[2][0] + parts[3][0])
                acc1 = (parts[0][1] + parts[1][1]) + (parts[2][1] + parts[3][1])
                out_v[ob + ql * NH + hh, pl.ds(0, 16)] = acc0
                out_v[ob + ql * NH + hh, pl.ds(16, 16)] = acc1
            return carry

        lax.fori_loop(0, NQC, obody, 0)

    idx_cp(0, 0).start()
    wgt_cp(0, 0).start()
    idx_cp(0, 0).wait()
    fire_gathers(0)
    idx_cp(1, 1).start()
    wgt_cp(1, 1).start()

    def pair(i, carry):
        t = i * 2
        idx_cp(t + 1, 1).wait()
        fire_gathers(1)
        wait_rows(0)

        @pl.when(t + 2 < NCHUNK)
        def _():
            idx_cp(t + 2, 0).start()

        wgt_cp(t, 0).wait()
        compute(0)

        @pl.when(t + 2 < NCHUNK)
        def _():
            wgt_cp(t + 2, 0).start()

        @pl.when(t + 2 < NCHUNK)
        def _():
            idx_cp(t + 2, 0).wait()
            fire_gathers(0)

        wait_rows(1)

        @pl.when(t + 3 < NCHUNK)
        def _():
            idx_cp(t + 3, 1).start()

        wgt_cp(t + 1, 1).wait()
        compute(1)
        orb = (wid * RPW + t * NQC) * NH
        pltpu.sync_copy(out_v, out_hbm.at[pl.ds(orb, 2 * NQC * NH)])

        @pl.when(t + 3 < NCHUNK)
        def _():
            wgt_cp(t + 3, 1).start()

        return carry

    lax.fori_loop(0, NCHUNK // 2, pair, 0)


def _sc_call(table, idx_flat, wgt_flat):
    mesh = plsc.VectorSubcoreMesh(core_axis_name="c", subcore_axis_name="s")
    idx2 = idx_flat.reshape(-1, 128)
    wgt2 = wgt_flat.reshape(-1, 128)
    fn = functools.partial(
        pl.kernel,
        mesh=mesh,
        out_type=jax.ShapeDtypeStruct((NROWS * NH, DH), jnp.float32),
        compiler_params=pltpu.CompilerParams(
            needs_layout_passes=False, use_tc_tiling_on_sc=False),
        scratch_types=[
            pltpu.VMEM((CE // 128, 128), jnp.int32),
            pltpu.VMEM((CE // 128, 128), jnp.int32),
            pltpu.VMEM((CE // 128, 128), jnp.float32),
            pltpu.VMEM((CE // 128, 128), jnp.float32),
            pltpu.VMEM((CE, DH), jnp.bfloat16),
            pltpu.VMEM((CE, DH), jnp.bfloat16),
            pltpu.VMEM((2 * NQC * NH, DH), jnp.float32),
            pltpu.SemaphoreType.DMA,
            pltpu.SemaphoreType.DMA,
            pltpu.SemaphoreType.DMA,
            pltpu.SemaphoreType.DMA,
            pltpu.SemaphoreType.DMA,
            pltpu.SemaphoreType.DMA,
        ],
    )(_sc_body)
    return fn(table, idx2, wgt2)


def _fuse_body(attn_ref, mask_ref, w_ref, b_ref, out_ref):
    flags = (jnp.sum(mask_ref[...], axis=2) > 0).astype(jnp.float32)
    slots = jnp.sum(attn_ref[...] * flags[:, :, None], axis=0)
    cnt = jnp.maximum(jnp.sum(flags, axis=0), 1.0)
    slots = slots / cnt[:, None]
    out_ref[...] = (
        jnp.dot(slots, w_ref[...], preferred_element_type=jnp.float32) + b_ref[...]
    )


def _fuse_call(attn, maskf, out_W, out_b):
    nqb = QLEN // QB
    return pl.pallas_call(
        _fuse_body,
        grid=(nqb,),
        in_specs=[
            pl.BlockSpec((NCAM, QB, DM), lambda q: (0, q, 0)),
            pl.BlockSpec((NCAM, QB, ZC), lambda q: (0, q, 0)),
            pl.BlockSpec((DM, DM), lambda q: (0, 0)),
            pl.BlockSpec((1, DM), lambda q: (0, 0)),
        ],
        out_specs=pl.BlockSpec((QB, DM), lambda q: (q, 0)),
        out_shape=jax.ShapeDtypeStruct((QLEN, DM), jnp.float32),
    )(attn, maskf, out_W, out_b)


def kernel(queries, pos_emb, lvl_emb, cam_emb, features, reference_points_3D,
           bev_mask, value_W, value_b, off_W, off_b, attw_W, attw_b,
           out_W, out_b):
    dperm = jnp.stack([jnp.arange(16), jnp.arange(16) + 16], axis=1).reshape(32)
    vw_p = value_W.reshape(DM, NH, DH)[:, :, dperm].reshape(DM, DM)
    vb_p = value_b.reshape(NH, DH)[:, dperm].reshape(1, DM)
    feat_t = jnp.transpose(features.reshape(NCAM, DM, HW), (0, 2, 1))
    value = _value_call(feat_t, lvl_emb, cam_emb, vw_p, vb_p)

    ow_p = off_W.reshape(DM, NH, NPT, 2).transpose(0, 3, 1, 2).reshape(DM, 128)
    ob_p = off_b.reshape(NH, NPT, 2).transpose(2, 0, 1).reshape(1, 128)
    q6 = queries.reshape(NCAM, QLEN, DM)
    pos = pos_emb.reshape(QLEN, DM)
    r = reference_points_3D.reshape(NCAM, QLEN, ZC, 2)
    rx64 = jnp.broadcast_to(
        r[:, :, None, None, :, 0], (NCAM, QLEN, NH, 2, ZC)).reshape(NCAM, QLEN, 64)
    ry64 = jnp.broadcast_to(
        r[:, :, None, None, :, 1], (NCAM, QLEN, NH, 2, ZC)).reshape(NCAM, QLEN, 64)
    idx, wgt = _prep_call(q6, pos, ow_p, ob_p, attw_W, attw_b.reshape(1, 64),
                          rx64, ry64)

    table = value.reshape(TROWS, DH)
    attn_flat = _sc_call(table, idx.reshape(-1), wgt.reshape(-1))
    attn = attn_flat.reshape(NCAM, QLEN, DM)

    maskf = bev_mask.reshape(NCAM, QLEN, ZC).astype(jnp.float32)
    out = _fuse_call(attn, maskf, out_W, out_b.reshape(1, DM))
    return out.reshape(1, QLEN, DM)
